# Initial kernel scaffold; baseline (speedup 1.0000x reference)
import jax
import jax.numpy as jnp
from jax.experimental import pallas as pl

TOP_K = 4096
REFINE_WIN = 5
REFINE_TAU = 0.2
CAND = 16384


def _conv8(x, w):
    return jax.lax.conv_general_dilated(x, w, (8, 8), 'VALID',
                                        dimension_numbers=('NCHW', 'OIHW', 'NCHW'))


def _cubic_w(t, a=-0.75):
    at = jnp.abs(t)
    return jnp.where(at <= 1.0, (a + 2) * at**3 - (a + 3) * at**2 + 1.0,
                     jnp.where(at < 2.0, a * at**3 - 5 * a * at**2 + 8 * a * at - 4 * a, 0.0))


def kernel(images, W_feat, W_kpts, W_heat):
    b, v, c, h, w = images.shape
    B = b * v
    x = images.reshape(B, c, h, w)

    kl = _conv8(x, W_kpts)
    hm_small = jax.nn.sigmoid(_conv8(x, W_heat))
    s = jax.nn.softmax(kl, axis=1)[:, :64]
    _, _, H8, W8 = s.shape
    khm = jnp.transpose(s, (0, 2, 3, 1)).reshape(B, H8, W8, 8, 8)
    khm = jnp.transpose(khm, (0, 1, 3, 2, 4)).reshape(B, 1, H8 * 8, W8 * 8)

    feats_map = _conv8(x, W_feat)

    lm = jax.lax.reduce_window(khm, -jnp.inf, jax.lax.max, (1, 1, 5, 5), (1, 1, 1, 1), 'SAME')
    mask = (khm == lm) & (khm > 0.0)
    H, W = h, w
    flat = mask[:, 0].reshape(B, H * W)
    key = jnp.where(flat, jnp.arange(1, H * W + 1, dtype=jnp.int32)[None, :], 0)
    topv, _ = jax.lax.top_k(key, CAND)
    cidx = jnp.maximum(topv - 1, 0)
    cx = cidx % W
    cy = cidx // W

    half = REFINE_WIN // 2
    hp = jnp.pad(khm[:, 0], ((0, 0), (half, half), (half, half)))
    r = jnp.arange(REFINE_WIN)
    iy = cy[..., None, None] + r[None, None, :, None]
    ix = cx[..., None, None] + r[None, None, None, :]
    bidx = jnp.arange(B)[:, None, None, None]
    patches = hp[bidx, iy, ix]
    logits = (patches / REFINE_TAU).reshape(B, -1, REFINE_WIN * REFINE_WIN)
    p = jax.nn.softmax(logits, axis=-1).reshape(B, -1, REFINE_WIN, REFINE_WIN)
    offs = jnp.arange(REFINE_WIN, dtype=jnp.float32) - half
    ex = (p * offs[None, None, None, :]).sum((-1, -2))
    ey = (p * offs[None, None, :, None]).sum((-1, -2))
    zero = (cidx == 0)
    rx = jnp.where(zero, 0.0, cx.astype(jnp.float32) + ex)
    ry = jnp.where(zero, 0.0, cy.astype(jnp.float32) + ey)

    def pix(px_, py_, Hf, Wf):
        gx = 2.0 * px_ / (W - 1) - 1.0
        gy = 2.0 * py_ / (H - 1) - 1.0
        return ((gx + 1.0) * Wf - 1.0) / 2.0, ((gy + 1.0) * Hf - 1.0) / 2.0

    def gz(xmap, iy_, ix_):
        Hf, Wf = xmap.shape[1], xmap.shape[2]
        valid = (iy_ >= 0) & (iy_ < Hf) & (ix_ >= 0) & (ix_ < Wf)
        iyc = jnp.clip(iy_, 0, Hf - 1)
        ixc = jnp.clip(ix_, 0, Wf - 1)
        vals = xmap[jnp.arange(B)[:, None], iyc, ixc]
        return vals * valid

    px, py = pix(rx, ry, H, W)
    s_n = gz(khm[:, 0], jnp.round(py).astype(jnp.int32), jnp.round(px).astype(jnp.int32))
    pxs, pys = pix(rx, ry, H8, W8)
    x0 = jnp.floor(pxs).astype(jnp.int32)
    y0 = jnp.floor(pys).astype(jnp.int32)
    dx = pxs - x0
    dy = pys - y0
    hs = hm_small[:, 0]
    v00 = gz(hs, y0, x0)
    v01 = gz(hs, y0, x0 + 1)
    v10 = gz(hs, y0 + 1, x0)
    v11 = gz(hs, y0 + 1, x0 + 1)
    s_b = v00 * (1 - dx) * (1 - dy) + v01 * dx * (1 - dy) + v10 * (1 - dx) * dy + v11 * dx * dy
    scores = jnp.where(zero, -1.0, s_n * s_b)

    svals, sidx = jax.lax.top_k(scores, TOP_K)
    kx = jnp.take_along_axis(rx, sidx, -1)
    ky = jnp.take_along_axis(ry, sidx, -1)
    kpts = jnp.stack([kx, ky], -1)

    fx_, fy_ = pix(kx, ky, H8, W8)
    fx0 = jnp.floor(fx_).astype(jnp.int32)
    fy0 = jnp.floor(fy_).astype(jnp.int32)
    fxf = fx_ - fx0
    fyf = fy_ - fy0
    fm = jnp.transpose(feats_map, (0, 2, 3, 1))

    def gzc(iy_, ix_):
        Hf, Wf = fm.shape[1], fm.shape[2]
        valid = (iy_ >= 0) & (iy_ < Hf) & (ix_ >= 0) & (ix_ < Wf)
        iyc = jnp.clip(iy_, 0, Hf - 1)
        ixc = jnp.clip(ix_, 0, Wf - 1)
        vals = fm[jnp.arange(B)[:, None], iyc, ixc]
        return vals * valid[..., None]

    acc = 0.0
    for m in range(-1, 3):
        wy = _cubic_w(fyf - m)[..., None]
        row = 0.0
        for n in range(-1, 3):
            wx = _cubic_w(fxf - n)[..., None]
            row = row + wx * gzc(fy0 + m, fx0 + n)
        acc = acc + wy * row
    feats = acc / jnp.clip(jnp.linalg.norm(acc, axis=-1, keepdims=True), 1e-12)
    return feats, svals, kpts


# speedup vs baseline: 53.0431x; 53.0431x over previous
import jax
import jax.numpy as jnp
from jax.experimental import pallas as pl

TOP_K = 4096
REFINE_WIN = 5
REFINE_TAU = 0.2
CAND = 16384


def _conv8(x, w):
    return jax.lax.conv_general_dilated(x, w, (8, 8), 'VALID',
                                        dimension_numbers=('NCHW', 'OIHW', 'NCHW'))


def _cubic_w(t, a=-0.75):
    at = jnp.abs(t)
    return jnp.where(at <= 1.0, (a + 2) * at**3 - (a + 3) * at**2 + 1.0,
                     jnp.where(at < 2.0, a * at**3 - 5 * a * at**2 + 8 * a * at - 4 * a, 0.0))


def kernel(images, W_feat, W_kpts, W_heat):
    b, v, c, h, w = images.shape
    B = b * v
    x = images.reshape(B, c, h, w)

    kl = _conv8(x, W_kpts)
    hm_small = jax.nn.sigmoid(_conv8(x, W_heat))
    s = jax.nn.softmax(kl, axis=1)[:, :64]
    _, _, H8, W8 = s.shape
    khm = jnp.transpose(s, (0, 2, 3, 1)).reshape(B, H8, W8, 8, 8)
    khm = jnp.transpose(khm, (0, 1, 3, 2, 4)).reshape(B, 1, H8 * 8, W8 * 8)

    feats_map = _conv8(x, W_feat)

    lm = jax.lax.reduce_window(khm, -jnp.inf, jax.lax.max, (1, 1, 5, 5), (1, 1, 1, 1), 'SAME')
    mask = (khm == lm) & (khm > 0.0)
    H, W = h, w
    flat = mask[:, 0].reshape(B, H * W)
    key = jnp.where(flat, jnp.arange(1, H * W + 1, dtype=jnp.int32)[None, :], 0)
    topv, _ = jax.lax.top_k(key, CAND)
    cidx = jnp.maximum(jnp.flip(topv, axis=-1) - 1, 0)
    cx = cidx % W
    cy = cidx // W

    half = REFINE_WIN // 2
    hp = jnp.pad(khm[:, 0], ((0, 0), (half, half), (half, half)))
    r = jnp.arange(REFINE_WIN)
    iy = cy[..., None, None] + r[None, None, :, None]
    ix = cx[..., None, None] + r[None, None, None, :]
    bidx = jnp.arange(B)[:, None, None, None]
    patches = hp[bidx, iy, ix]
    logits = (patches / REFINE_TAU).reshape(B, -1, REFINE_WIN * REFINE_WIN)
    p = jax.nn.softmax(logits, axis=-1).reshape(B, -1, REFINE_WIN, REFINE_WIN)
    offs = jnp.arange(REFINE_WIN, dtype=jnp.float32) - half
    ex = (p * offs[None, None, None, :]).sum((-1, -2))
    ey = (p * offs[None, None, :, None]).sum((-1, -2))
    zero = (cidx == 0)
    rx = jnp.where(zero, 0.0, cx.astype(jnp.float32) + ex)
    ry = jnp.where(zero, 0.0, cy.astype(jnp.float32) + ey)

    def pix(px_, py_, Hf, Wf):
        gx = 2.0 * px_ / (W - 1) - 1.0
        gy = 2.0 * py_ / (H - 1) - 1.0
        return ((gx + 1.0) * Wf - 1.0) / 2.0, ((gy + 1.0) * Hf - 1.0) / 2.0

    def gz(xmap, iy_, ix_):
        Hf, Wf = xmap.shape[1], xmap.shape[2]
        valid = (iy_ >= 0) & (iy_ < Hf) & (ix_ >= 0) & (ix_ < Wf)
        iyc = jnp.clip(iy_, 0, Hf - 1)
        ixc = jnp.clip(ix_, 0, Wf - 1)
        vals = xmap[jnp.arange(B)[:, None], iyc, ixc]
        return vals * valid

    px, py = pix(rx, ry, H, W)
    s_n = gz(khm[:, 0], jnp.round(py).astype(jnp.int32), jnp.round(px).astype(jnp.int32))
    pxs, pys = pix(rx, ry, H8, W8)
    x0 = jnp.floor(pxs).astype(jnp.int32)
    y0 = jnp.floor(pys).astype(jnp.int32)
    dx = pxs - x0
    dy = pys - y0
    hs = hm_small[:, 0]
    v00 = gz(hs, y0, x0)
    v01 = gz(hs, y0, x0 + 1)
    v10 = gz(hs, y0 + 1, x0)
    v11 = gz(hs, y0 + 1, x0 + 1)
    s_b = v00 * (1 - dx) * (1 - dy) + v01 * dx * (1 - dy) + v10 * (1 - dx) * dy + v11 * dx * dy
    scores = jnp.where(zero, -1.0, s_n * s_b)

    svals, sidx = jax.lax.top_k(scores, TOP_K)
    kx = jnp.take_along_axis(rx, sidx, -1)
    ky = jnp.take_along_axis(ry, sidx, -1)
    kpts = jnp.stack([kx, ky], -1)

    fx_, fy_ = pix(kx, ky, H8, W8)
    fx0 = jnp.floor(fx_).astype(jnp.int32)
    fy0 = jnp.floor(fy_).astype(jnp.int32)
    fxf = fx_ - fx0
    fyf = fy_ - fy0
    fm = jnp.transpose(feats_map, (0, 2, 3, 1))

    def gzc(iy_, ix_):
        Hf, Wf = fm.shape[1], fm.shape[2]
        valid = (iy_ >= 0) & (iy_ < Hf) & (ix_ >= 0) & (ix_ < Wf)
        iyc = jnp.clip(iy_, 0, Hf - 1)
        ixc = jnp.clip(ix_, 0, Wf - 1)
        vals = fm[jnp.arange(B)[:, None], iyc, ixc]
        return vals * valid[..., None]

    acc = 0.0
    for m in range(-1, 3):
        wy = _cubic_w(fyf - m)[..., None]
        row = 0.0
        for n in range(-1, 3):
            wx = _cubic_w(fxf - n)[..., None]
            row = row + wx * gzc(fy0 + m, fx0 + n)
        acc = acc + wy * row
    feats = acc / jnp.clip(jnp.linalg.norm(acc, axis=-1, keepdims=True), 1e-12)
    return feats, svals, kpts


# speedup vs baseline: 62.8435x; 1.1848x over previous
import functools

import jax
import jax.numpy as jnp
from jax import lax
from jax.experimental import pallas as pl
from jax.experimental.pallas import tpu as pltpu
from jax.experimental.pallas import tpu_sc as plsc

TOP_K = 4096
REFINE_WIN = 5
REFINE_TAU = 0.2
STRIP_CAP = 2048
CAND = 8 * STRIP_CAP
H_IMG = 512
W_IMG = 512


def _hmax_body(khm_ref, out_ref):
    x = khm_ref[...]
    m = x
    for s in (1, 2):
        z = jnp.zeros((x.shape[0], s), dtype=x.dtype)
        left = jnp.concatenate([z, x[:, :-s]], axis=1)
        right = jnp.concatenate([x[:, s:], z], axis=1)
        m = jnp.maximum(m, jnp.maximum(left, right))
    out_ref[...] = m


def _hmax_5(khm2d):
    return pl.pallas_call(
        _hmax_body,
        grid=(4,),
        in_specs=[pl.BlockSpec((H_IMG, W_IMG), lambda i: (i, 0))],
        out_specs=pl.BlockSpec((H_IMG, W_IMG), lambda i: (i, 0)),
        out_shape=jax.ShapeDtypeStruct((4 * H_IMG, W_IMG), jnp.float32),
    )(khm2d)


def _nms_compact_sc(khm2d, hmax2d):
    mesh = plsc.VectorSubcoreMesh(core_axis_name="c", subcore_axis_name="s")
    W = W_IMG

    @functools.partial(
        pl.kernel,
        mesh=mesh,
        compiler_params=pltpu.CompilerParams(needs_layout_passes=False),
        out_type=jax.ShapeDtypeStruct((32 * STRIP_CAP,), jnp.int32),
        scratch_types=[
            pltpu.VMEM((64 * W,), jnp.float32),
            pltpu.VMEM((68 * W,), jnp.float32),
            pltpu.VMEM((STRIP_CAP + 16,), jnp.int32),
        ],
    )
    def body(khm_hbm, hmax_hbm, out_hbm, khm_v, hmax_v, cand_v):
        wid = lax.axis_index("s") * 2 + lax.axis_index("c")
        img = wid // 8
        strip = wid % 8
        base = strip * 64
        src0 = (img * H_IMG + base) * W

        zeros16i = jnp.zeros((16,), jnp.int32)
        zeros16f = jnp.zeros((16,), jnp.float32)

        def zero_cand(i, _):
            cand_v[pl.ds(i * 16, 16)] = zeros16i
            return 0
        lax.fori_loop(0, (STRIP_CAP + 16) // 16, zero_cand, 0)

        pltpu.sync_copy(khm_hbm.at[pl.ds(src0, 64 * W)], khm_v.at[:])

        @pl.when(strip == 0)
        def _():
            def zrow(i, _):
                hmax_v[pl.ds(i * 16, 16)] = zeros16f
                return 0
            lax.fori_loop(0, 2 * W // 16, zrow, 0)
            pltpu.sync_copy(hmax_hbm.at[pl.ds(src0, 66 * W)], hmax_v.at[pl.ds(2 * W, 66 * W)])

        @pl.when(strip == 7)
        def _():
            def zrow(i, _):
                hmax_v[pl.ds(66 * W + i * 16, 16)] = zeros16f
                return 0
            lax.fori_loop(0, 2 * W // 16, zrow, 0)
            pltpu.sync_copy(hmax_hbm.at[pl.ds(src0 - 2 * W, 66 * W)], hmax_v.at[pl.ds(0, 66 * W)])

        @pl.when(jnp.logical_and(strip != 0, strip != 7))
        def _():
            pltpu.sync_copy(hmax_hbm.at[pl.ds(src0 - 2 * W, 68 * W)], hmax_v.at[pl.ds(0, 68 * W)])

        lane = lax.iota(jnp.int32, 16)
        flat0 = base * W

        def row_body(r, off):
            def chunk_body(c, off):
                col = c * 16
                p = r * W + col
                v = khm_v[pl.ds(p, 16)]
                m = hmax_v[pl.ds(p, 16)]
                m = jnp.maximum(m, hmax_v[pl.ds(p + W, 16)])
                m = jnp.maximum(m, hmax_v[pl.ds(p + 2 * W, 16)])
                m = jnp.maximum(m, hmax_v[pl.ds(p + 3 * W, 16)])
                m = jnp.maximum(m, hmax_v[pl.ds(p + 4 * W, 16)])
                mask = jnp.logical_and(v == m, v > 0.0)
                idx = (flat0 + p) + lane
                cnt = jnp.sum(mask.astype(jnp.int32))

                @pl.when(off < STRIP_CAP)
                def _():
                    plsc.store_compressed(cand_v.at[pl.ds(off, 16)], idx, mask=mask)

                return off + cnt
            return lax.fori_loop(0, W // 16, chunk_body, off)

        lax.fori_loop(0, 64, row_body, 0)
        pltpu.sync_copy(cand_v.at[pl.ds(0, STRIP_CAP)],
                        out_hbm.at[pl.ds(wid * STRIP_CAP, STRIP_CAP)])

    return body(khm2d.reshape(-1), hmax2d.reshape(-1))


def _conv8(x, w):
    return jax.lax.conv_general_dilated(x, w, (8, 8), 'VALID',
                                        dimension_numbers=('NCHW', 'OIHW', 'NCHW'))


def _cubic_w(t, a=-0.75):
    at = jnp.abs(t)
    return jnp.where(at <= 1.0, (a + 2) * at**3 - (a + 3) * at**2 + 1.0,
                     jnp.where(at < 2.0, a * at**3 - 5 * a * at**2 + 8 * a * at - 4 * a, 0.0))


def kernel(images, W_feat, W_kpts, W_heat):
    b, v, c, h, w = images.shape
    B = b * v
    x = images.reshape(B, c, h, w)

    kl = _conv8(x, W_kpts)
    hm_small = jax.nn.sigmoid(_conv8(x, W_heat))
    s = jax.nn.softmax(kl, axis=1)[:, :64]
    _, _, H8, W8 = s.shape
    khm = jnp.transpose(s, (0, 2, 3, 1)).reshape(B, H8, W8, 8, 8)
    khm = jnp.transpose(khm, (0, 1, 3, 2, 4)).reshape(B, 1, H8 * 8, W8 * 8)

    feats_map = _conv8(x, W_feat)

    H, W = h, w
    khm2d = khm[:, 0].reshape(B * H, W)
    hmax2d = _hmax_5(khm2d)
    cand = _nms_compact_sc(khm2d, hmax2d)
    cidx = cand.reshape(B, CAND)
    cx = cidx % W
    cy = cidx // W

    half = REFINE_WIN // 2
    hp = jnp.pad(khm[:, 0], ((0, 0), (half, half), (half, half)))
    r = jnp.arange(REFINE_WIN)
    iy = cy[..., None, None] + r[None, None, :, None]
    ix = cx[..., None, None] + r[None, None, None, :]
    bidx = jnp.arange(B)[:, None, None, None]
    patches = hp[bidx, iy, ix]
    logits = (patches / REFINE_TAU).reshape(B, -1, REFINE_WIN * REFINE_WIN)
    p = jax.nn.softmax(logits, axis=-1).reshape(B, -1, REFINE_WIN, REFINE_WIN)
    offs = jnp.arange(REFINE_WIN, dtype=jnp.float32) - half
    ex = (p * offs[None, None, None, :]).sum((-1, -2))
    ey = (p * offs[None, None, :, None]).sum((-1, -2))
    zero = (cidx == 0)
    rx = jnp.where(zero, 0.0, cx.astype(jnp.float32) + ex)
    ry = jnp.where(zero, 0.0, cy.astype(jnp.float32) + ey)

    def pix(px_, py_, Hf, Wf):
        gx = 2.0 * px_ / (W - 1) - 1.0
        gy = 2.0 * py_ / (H - 1) - 1.0
        return ((gx + 1.0) * Wf - 1.0) / 2.0, ((gy + 1.0) * Hf - 1.0) / 2.0

    def gz(xmap, iy_, ix_):
        Hf, Wf = xmap.shape[1], xmap.shape[2]
        valid = (iy_ >= 0) & (iy_ < Hf) & (ix_ >= 0) & (ix_ < Wf)
        iyc = jnp.clip(iy_, 0, Hf - 1)
        ixc = jnp.clip(ix_, 0, Wf - 1)
        vals = xmap[jnp.arange(B)[:, None], iyc, ixc]
        return vals * valid

    px, py = pix(rx, ry, H, W)
    s_n = gz(khm[:, 0], jnp.round(py).astype(jnp.int32), jnp.round(px).astype(jnp.int32))
    pxs, pys = pix(rx, ry, H8, W8)
    x0 = jnp.floor(pxs).astype(jnp.int32)
    y0 = jnp.floor(pys).astype(jnp.int32)
    dx = pxs - x0
    dy = pys - y0
    hs = hm_small[:, 0]
    v00 = gz(hs, y0, x0)
    v01 = gz(hs, y0, x0 + 1)
    v10 = gz(hs, y0 + 1, x0)
    v11 = gz(hs, y0 + 1, x0 + 1)
    s_b = v00 * (1 - dx) * (1 - dy) + v01 * dx * (1 - dy) + v10 * (1 - dx) * dy + v11 * dx * dy
    scores = jnp.where(zero, -1.0, s_n * s_b)

    svals, sidx = jax.lax.top_k(scores, TOP_K)
    kx = jnp.take_along_axis(rx, sidx, -1)
    ky = jnp.take_along_axis(ry, sidx, -1)
    kpts = jnp.stack([kx, ky], -1)

    fx_, fy_ = pix(kx, ky, H8, W8)
    fx0 = jnp.floor(fx_).astype(jnp.int32)
    fy0 = jnp.floor(fy_).astype(jnp.int32)
    fxf = fx_ - fx0
    fyf = fy_ - fy0
    fm = jnp.transpose(feats_map, (0, 2, 3, 1))

    def gzc(iy_, ix_):
        Hf, Wf = fm.shape[1], fm.shape[2]
        valid = (iy_ >= 0) & (iy_ < Hf) & (ix_ >= 0) & (ix_ < Wf)
        iyc = jnp.clip(iy_, 0, Hf - 1)
        ixc = jnp.clip(ix_, 0, Wf - 1)
        vals = fm[jnp.arange(B)[:, None], iyc, ixc]
        return vals * valid[..., None]

    acc = 0.0
    for m in range(-1, 3):
        wy = _cubic_w(fyf - m)[..., None]
        row = 0.0
        for n in range(-1, 3):
            wx = _cubic_w(fxf - n)[..., None]
            row = row + wx * gzc(fy0 + m, fx0 + n)
        acc = acc + wy * row
    feats = acc / jnp.clip(jnp.linalg.norm(acc, axis=-1, keepdims=True), 1e-12)
    return feats, svals, kpts
